# Initial kernel scaffold; baseline (speedup 1.0000x reference)
import jax
import jax.numpy as jnp
from jax.experimental import pallas as pl


def kernel(x, edge_index, pos_edges, neg_edges, W1, b1, W2, b2, Wd1, bd1, Wd2, bd2):
    raise NotImplementedError("write your pallas kernel here")


# speedup vs baseline: 16.4167x; 16.4167x over previous
import functools

import jax
import jax.numpy as jnp
from jax import lax
from jax.experimental import pallas as pl
from jax.experimental.pallas import tpu as pltpu
from jax.experimental.pallas import tpu_sc as plsc

N = 10000
D = 128
E = 320000
EDEC = 200000

NC, NS, L = 2, 16, 16
NW = NC * NS

EB = 125
E_NB = E // (NW * EB)
DEC_NB = EDEC // (NW * EB)

NPAD = 10240
STRIPE = NPAD // NS
DSTRIPE = NPAD // NS

_MESH = plsc.VectorSubcoreMesh(
    core_axis_name="c", subcore_axis_name="s", num_cores=NC, num_subcores=NS)

_f32 = jnp.float32


def _zero16():
    return jnp.zeros((L,), _f32)


@functools.partial(
    pl.kernel,
    out_type=jax.ShapeDtypeStruct((NC, 1, NPAD), _f32),
    mesh=_MESH,
    scratch_types=[
        pltpu.VMEM((E_NB, EB), jnp.int32),
        pltpu.VMEM((128,), _f32),
        pltpu.VMEM((DSTRIPE,), _f32),
        pltpu.VMEM_SHARED((NPAD,), _f32),
    ],
)
def _deg(dst3, out, idx_v, ones_v, zstripe, acc_sh):
    c = lax.axis_index("c")
    s = lax.axis_index("s")
    for k in range(8):
        ones_v[pl.ds(k * L, L)] = jnp.full((L,), 1.0, _f32)
    for k in range(DSTRIPE // L):
        zstripe[pl.ds(k * L, L)] = _zero16()
    pltpu.sync_copy(dst3.at[c, s], idx_v)
    pltpu.sync_copy(zstripe, acc_sh.at[pl.ds(s * DSTRIPE, DSTRIPE)])
    plsc.subcore_barrier()

    def body(j, carry):
        pltpu.sync_copy(ones_v.at[pl.ds(0, EB)], acc_sh.at[idx_v.at[j]], add=True)
        return carry

    lax.fori_loop(0, E_NB, body, 0)
    plsc.subcore_barrier()
    pltpu.sync_copy(acc_sh.at[pl.ds(s * DSTRIPE, DSTRIPE)],
                    out.at[c, 0, pl.ds(s * DSTRIPE, DSTRIPE)])


@functools.partial(
    pl.kernel,
    out_type=jax.ShapeDtypeStruct((NC, NPAD, D), _f32),
    mesh=_MESH,
    scratch_types=[
        pltpu.VMEM((E_NB // 2, EB), jnp.int32),
        pltpu.VMEM((E_NB // 2, EB), jnp.int32),
        pltpu.VMEM((EB, D), _f32),
        pltpu.VMEM((EB, D), _f32),
        pltpu.VMEM_SHARED((NPAD, D), _f32),
        pltpu.SemaphoreType.DMA,
        pltpu.SemaphoreType.DMA,
    ],
)
def _scatter(hs, src3, dst3, out, sidx, didx, rows0, rows1, acc_sh, sem0, sem1):
    c = lax.axis_index("c")
    s = lax.axis_index("s")
    nb = E_NB // 2

    def zrow(i, carry):
        for r in range(D // L):
            rows0[i, pl.ds(r * L, L)] = _zero16()
        return carry

    lax.fori_loop(0, EB, zrow, 0)
    for t in range(STRIPE // EB):
        pltpu.sync_copy(rows0, acc_sh.at[pl.ds(s * STRIPE + t * EB, EB)])
    rem = STRIPE - (STRIPE // EB) * EB
    pltpu.sync_copy(rows0.at[pl.ds(0, rem)],
                    acc_sh.at[pl.ds(s * STRIPE + (STRIPE // EB) * EB, rem)])
    plsc.subcore_barrier()

    for p in range(2):
        pltpu.sync_copy(src3.at[c, s, pl.ds(p * nb, nb)], sidx)
        pltpu.sync_copy(dst3.at[c, s, pl.ds(p * nb, nb)], didx)
        pltpu.async_copy(hs.at[sidx.at[0]], rows0, sem0).wait()

        def body(j, carry):
            b = 2 * j
            nxt = pltpu.async_copy(hs.at[sidx.at[b + 1]], rows1, sem1)
            pltpu.sync_copy(rows0, acc_sh.at[didx.at[b]], add=True)
            nxt.wait()
            cp = pltpu.async_copy(hs.at[sidx.at[b + 2]], rows0, sem0)
            pltpu.sync_copy(rows1, acc_sh.at[didx.at[b + 1]], add=True)
            cp.wait()
            return carry

        lax.fori_loop(0, (nb - 2) // 2, body, 0)
        nxt = pltpu.async_copy(hs.at[sidx.at[nb - 1]], rows1, sem1)
        pltpu.sync_copy(rows0, acc_sh.at[didx.at[nb - 2]], add=True)
        nxt.wait()
        pltpu.sync_copy(rows1, acc_sh.at[didx.at[nb - 1]], add=True)

    plsc.subcore_barrier()
    pltpu.sync_copy(acc_sh.at[pl.ds(s * STRIPE, STRIPE)],
                    out.at[c, pl.ds(s * STRIPE, STRIPE)])


@functools.partial(
    pl.kernel,
    out_type=jax.ShapeDtypeStruct((NC, NS, DEC_NB, EB, L), _f32),
    mesh=_MESH,
    scratch_types=[
        pltpu.VMEM((DEC_NB, EB), jnp.int32),
        pltpu.VMEM((DEC_NB, EB), jnp.int32),
        pltpu.VMEM((EB, D), _f32),
        pltpu.VMEM((EB, D), _f32),
        pltpu.VMEM((EB, D), _f32),
        pltpu.VMEM((EB, D), _f32),
        pltpu.VMEM((D,), _f32),
        pltpu.VMEM((D,), _f32),
        pltpu.VMEM((EB, L), _f32),
        pltpu.VMEM((EB, L), _f32),
        pltpu.SemaphoreType.DMA,
        pltpu.SemaphoreType.DMA,
    ],
)
def _dec(zwu, zwv, u3, v3, bd1, wd2, out,
         uidx, vidx, ra0, rb0, ra1, rb1, bd1_v, wd2_v, ob0, ob1, semA, semB):
    c = lax.axis_index("c")
    s = lax.axis_index("s")
    pltpu.sync_copy(u3.at[c, s], uidx)
    pltpu.sync_copy(v3.at[c, s], vidx)
    pltpu.sync_copy(bd1, bd1_v)
    pltpu.sync_copy(wd2, wd2_v)
    bd1r = [bd1_v[pl.ds(r * L, L)] for r in range(D // L)]
    wd2r = [wd2_v[pl.ds(r * L, L)] for r in range(D // L)]

    def compute(j, ra, rb, ob):
        def edge(e, carry):
            acc = _zero16()
            for r in range(D // L):
                g = ra[e, pl.ds(r * L, L)] + rb[e, pl.ds(r * L, L)] + bd1r[r]
                acc = acc + jnp.maximum(g, 0.0) * wd2r[r]
            ob[e] = acc
            return carry
        lax.fori_loop(0, EB, edge, 0)
        pltpu.sync_copy(ob, out.at[c, s, j])

    pltpu.async_copy(zwu.at[uidx.at[0]], ra0, semA).wait()
    pltpu.async_copy(zwv.at[vidx.at[0]], rb0, semB).wait()

    def body(j, carry):
        b = 2 * j
        na = pltpu.async_copy(zwu.at[uidx.at[b + 1]], ra1, semA)
        nb = pltpu.async_copy(zwv.at[vidx.at[b + 1]], rb1, semB)
        compute(b, ra0, rb0, ob0)
        na.wait()
        nb.wait()
        ca = pltpu.async_copy(zwu.at[uidx.at[b + 2]], ra0, semA)
        cb = pltpu.async_copy(zwv.at[vidx.at[b + 2]], rb0, semB)
        compute(b + 1, ra1, rb1, ob1)
        ca.wait()
        cb.wait()
        return carry

    lax.fori_loop(0, (DEC_NB - 2) // 2, body, 0)
    na = pltpu.async_copy(zwu.at[uidx.at[DEC_NB - 1]], ra1, semA)
    nb = pltpu.async_copy(zwv.at[vidx.at[DEC_NB - 1]], rb1, semB)
    compute(DEC_NB - 2, ra0, rb0, ob0)
    na.wait()
    nb.wait()
    compute(DEC_NB - 1, ra1, rb1, ob1)


_BR = 512
_GRID = (N + _BR - 1) // _BR


def _mm1_body(cnt_ref, x_ref, w_ref, hs_ref, dis_ref):
    deg = cnt_ref[0] + cnt_ref[1] + 1.0
    dis = lax.rsqrt(deg)
    h = jnp.dot(x_ref[...], w_ref[...], preferred_element_type=_f32,
                precision=lax.Precision.HIGHEST)
    hs_ref[...] = h * dis[:, None]
    dis_ref[...] = dis


def _mm2_body(acc_ref, hs_ref, dis_ref, b_ref, w_ref, out_ref, *, relu_in):
    dis = dis_ref[...]
    z = dis[:, None] * (acc_ref[0] + acc_ref[1] + hs_ref[...]) + b_ref[...][None, :]
    if relu_in:
        z = jnp.maximum(z, 0.0)
    h = jnp.dot(z, w_ref[...], preferred_element_type=_f32,
                precision=lax.Precision.HIGHEST)
    out_ref[...] = h * dis[:, None]


def _mm3_body(acc_ref, hs_ref, dis_ref, b_ref, w_ref, u_ref, v_ref):
    dis = dis_ref[...]
    z = dis[:, None] * (acc_ref[0] + acc_ref[1] + hs_ref[...]) + b_ref[...][None, :]
    w = w_ref[...]
    u_ref[...] = jnp.dot(z, w[:D], preferred_element_type=_f32,
                         precision=lax.Precision.HIGHEST)
    v_ref[...] = jnp.dot(z, w[D:], preferred_element_type=_f32,
                         precision=lax.Precision.HIGHEST)


def _tc_mm1(cnt, x, w):
    return pl.pallas_call(
        _mm1_body,
        grid=(_GRID,),
        in_specs=[
            pl.BlockSpec((2, _BR), lambda i: (0, i)),
            pl.BlockSpec((_BR, D), lambda i: (i, 0)),
            pl.BlockSpec((D, D), lambda i: (0, 0)),
        ],
        out_specs=[
            pl.BlockSpec((_BR, D), lambda i: (i, 0)),
            pl.BlockSpec((_BR,), lambda i: (i,)),
        ],
        out_shape=[
            jax.ShapeDtypeStruct((N, D), _f32),
            jax.ShapeDtypeStruct((N,), _f32),
        ],
    )(cnt, x, w)


def _tc_mm2(acc, hs, dis, b, w, relu_in):
    return pl.pallas_call(
        functools.partial(_mm2_body, relu_in=relu_in),
        grid=(_GRID,),
        in_specs=[
            pl.BlockSpec((2, _BR, D), lambda i: (0, i, 0)),
            pl.BlockSpec((_BR, D), lambda i: (i, 0)),
            pl.BlockSpec((_BR,), lambda i: (i,)),
            pl.BlockSpec((D,), lambda i: (0,)),
            pl.BlockSpec((D, D), lambda i: (0, 0)),
        ],
        out_specs=pl.BlockSpec((_BR, D), lambda i: (i, 0)),
        out_shape=jax.ShapeDtypeStruct((N, D), _f32),
    )(acc, hs, dis, b, w)


def _tc_mm3(acc, hs, dis, b, w):
    return pl.pallas_call(
        _mm3_body,
        grid=(_GRID,),
        in_specs=[
            pl.BlockSpec((2, _BR, D), lambda i: (0, i, 0)),
            pl.BlockSpec((_BR, D), lambda i: (i, 0)),
            pl.BlockSpec((_BR,), lambda i: (i,)),
            pl.BlockSpec((D,), lambda i: (0,)),
            pl.BlockSpec((2 * D, D), lambda i: (0, 0)),
        ],
        out_specs=[
            pl.BlockSpec((_BR, D), lambda i: (i, 0)),
            pl.BlockSpec((_BR, D), lambda i: (i, 0)),
        ],
        out_shape=[
            jax.ShapeDtypeStruct((N, D), _f32),
            jax.ShapeDtypeStruct((N, D), _f32),
        ],
    )(acc, hs, dis, b, w)


_RB = 8192
_RGRID = (EDEC + _RB - 1) // _RB


def _red_body(m_ref, b_ref, o_ref):
    o_ref[...] = jnp.sum(m_ref[...], axis=1) + b_ref[...][0]


def _tc_red(m, bd2):
    return pl.pallas_call(
        _red_body,
        grid=(_RGRID,),
        in_specs=[
            pl.BlockSpec((_RB, L), lambda i: (i, 0)),
            pl.BlockSpec((1,), lambda i: (0,)),
        ],
        out_specs=pl.BlockSpec((_RB,), lambda i: (i,)),
        out_shape=jax.ShapeDtypeStruct((EDEC,), _f32),
    )(m, bd2)


def kernel(x, edge_index, pos_edges, neg_edges, W1, b1, W2, b2, Wd1, bd1, Wd2, bd2):
    ei = edge_index.astype(jnp.int32)
    src3 = ei[0].reshape(NC, NS, E_NB, EB)
    dst3 = ei[1].reshape(NC, NS, E_NB, EB)
    u3 = jnp.concatenate([pos_edges[0], neg_edges[0]]).astype(jnp.int32)
    v3 = jnp.concatenate([pos_edges[1], neg_edges[1]]).astype(jnp.int32)
    u3 = u3.reshape(NC, NS, DEC_NB, EB)
    v3 = v3.reshape(NC, NS, DEC_NB, EB)

    cnt = _deg(dst3)[:, 0, :]
    hs1, dis = _tc_mm1(cnt, x, W1)
    acc1 = _scatter(hs1, src3, dst3)
    hs2 = _tc_mm2(acc1, hs1, dis, b1, W2, relu_in=True)
    acc2 = _scatter(hs2, src3, dst3)
    zwu, zwv = _tc_mm3(acc2, hs2, dis, b2, Wd1)

    wd2 = Wd2.reshape(D)
    outg = _dec(zwu, zwv, u3, v3, bd1, wd2)
    return _tc_red(outg.reshape(EDEC, L), bd2)


# speedup vs baseline: 17.2706x; 1.0520x over previous
import functools

import jax
import jax.numpy as jnp
from jax import lax
from jax.experimental import pallas as pl
from jax.experimental.pallas import tpu as pltpu
from jax.experimental.pallas import tpu_sc as plsc

N = 10000
D = 128
E = 320000
EDEC = 200000

NC, NS, L = 2, 16, 16
NW = NC * NS

EB = 125
E_NB = E // (NW * EB)
DEC_NB = EDEC // (NW * EB)

SB = 80
S_NB = E // (NW * SB)
S_PASS = 5
S_CHUNK = S_NB // S_PASS

NPAD = 10240
STRIPE = NPAD // NS
DSTRIPE = NPAD // NS

_MESH = plsc.VectorSubcoreMesh(
    core_axis_name="c", subcore_axis_name="s", num_cores=NC, num_subcores=NS)

_f32 = jnp.float32


def _zero16():
    return jnp.zeros((L,), _f32)


@functools.partial(
    pl.kernel,
    out_type=jax.ShapeDtypeStruct((NC, 1, NPAD), _f32),
    mesh=_MESH,
    scratch_types=[
        pltpu.VMEM((E_NB, EB), jnp.int32),
        pltpu.VMEM((128,), _f32),
        pltpu.VMEM((DSTRIPE,), _f32),
        pltpu.VMEM_SHARED((NPAD,), _f32),
    ],
)
def _deg(dst3, out, idx_v, ones_v, zstripe, acc_sh):
    c = lax.axis_index("c")
    s = lax.axis_index("s")
    for k in range(8):
        ones_v[pl.ds(k * L, L)] = jnp.full((L,), 1.0, _f32)
    for k in range(DSTRIPE // L):
        zstripe[pl.ds(k * L, L)] = _zero16()
    pltpu.sync_copy(dst3.at[c, s], idx_v)
    pltpu.sync_copy(zstripe, acc_sh.at[pl.ds(s * DSTRIPE, DSTRIPE)])
    plsc.subcore_barrier()

    def body(j, carry):
        pltpu.sync_copy(ones_v.at[pl.ds(0, EB)], acc_sh.at[idx_v.at[j]], add=True)
        return carry

    lax.fori_loop(0, E_NB, body, 0)
    plsc.subcore_barrier()
    pltpu.sync_copy(acc_sh.at[pl.ds(s * DSTRIPE, DSTRIPE)],
                    out.at[c, 0, pl.ds(s * DSTRIPE, DSTRIPE)])


@functools.partial(
    pl.kernel,
    out_type=jax.ShapeDtypeStruct((NC, NPAD, D), _f32),
    mesh=_MESH,
    scratch_types=[
        pltpu.VMEM((S_CHUNK, SB), jnp.int32),
        pltpu.VMEM((S_CHUNK, SB), jnp.int32),
        pltpu.VMEM((SB, D), _f32),
        pltpu.VMEM((SB, D), _f32),
        pltpu.VMEM((SB, D), _f32),
        pltpu.VMEM_SHARED((NPAD, D), _f32),
        pltpu.SemaphoreType.DMA,
        pltpu.SemaphoreType.DMA,
        pltpu.SemaphoreType.DMA,
        pltpu.SemaphoreType.DMA,
        pltpu.SemaphoreType.DMA,
        pltpu.SemaphoreType.DMA,
    ],
)
def _scatter(hs, src3, dst3, out, sidx, didx, b0, b1, b2, acc_sh,
             gs0, gs1, gs2, ss0, ss1, ss2):
    c = lax.axis_index("c")
    s = lax.axis_index("s")
    bufs = (b0, b1, b2)
    gsems = (gs0, gs1, gs2)
    ssems = (ss0, ss1, ss2)
    nb = S_CHUNK

    def zrow(i, carry):
        for r in range(D // L):
            b0[i, pl.ds(r * L, L)] = _zero16()
        return carry

    lax.fori_loop(0, SB, zrow, 0)
    for t in range(STRIPE // SB):
        pltpu.sync_copy(b0, acc_sh.at[pl.ds(s * STRIPE + t * SB, SB)])
    plsc.subcore_barrier()

    def wait_gather(t, b):
        pltpu.make_async_copy(hs.at[sidx.at[t]], bufs[b], gsems[b]).wait()

    def wait_scatter(b):
        pltpu.make_async_copy(bufs[b], acc_sh.at[didx.at[0]], ssems[b]).wait()

    def issue_gather(t, b):
        pltpu.async_copy(hs.at[sidx.at[t]], bufs[b], gsems[b])

    def issue_scatter(t, b):
        pltpu.async_copy(bufs[b], acc_sh.at[didx.at[t]], ssems[b], add=True)

    def slot(t, b, swait):
        wait_gather(t, b)
        issue_scatter(t, b)
        if swait:
            wait_scatter((b + 2) % 3)
        issue_gather(t + 2, (b + 2) % 3)

    for p in range(S_PASS):
        pltpu.sync_copy(src3.at[c, s, p], sidx)
        pltpu.sync_copy(dst3.at[c, s, p], didx)
        issue_gather(0, 0)
        issue_gather(1, 1)
        slot(0, 0, False)

        def body(j, carry):
            t = 3 * j + 1
            slot(t, 1, True)
            slot(t + 1, 2, True)
            slot(t + 2, 0, True)
            return carry

        lax.fori_loop(0, (nb - 4) // 3, body, 0)
        t0 = nb - 3
        slot(t0, 1, True)
        wait_gather(t0 + 1, 2)
        issue_scatter(t0 + 1, 2)
        wait_gather(t0 + 2, 0)
        issue_scatter(t0 + 2, 0)
        wait_scatter(1)
        wait_scatter(2)
        wait_scatter(0)

    plsc.subcore_barrier()
    pltpu.sync_copy(acc_sh.at[pl.ds(s * STRIPE, STRIPE)],
                    out.at[c, pl.ds(s * STRIPE, STRIPE)])


@functools.partial(
    pl.kernel,
    out_type=jax.ShapeDtypeStruct((NC, NS, DEC_NB, EB, L), _f32),
    mesh=_MESH,
    scratch_types=[
        pltpu.VMEM((DEC_NB, EB), jnp.int32),
        pltpu.VMEM((DEC_NB, EB), jnp.int32),
        pltpu.VMEM((EB, D), _f32),
        pltpu.VMEM((EB, D), _f32),
        pltpu.VMEM((EB, D), _f32),
        pltpu.VMEM((EB, D), _f32),
        pltpu.VMEM((D,), _f32),
        pltpu.VMEM((D,), _f32),
        pltpu.VMEM((EB, L), _f32),
        pltpu.VMEM((EB, L), _f32),
        pltpu.SemaphoreType.DMA,
        pltpu.SemaphoreType.DMA,
    ],
)
def _dec(zwu, zwv, u3, v3, bd1, wd2, out,
         uidx, vidx, ra0, rb0, ra1, rb1, bd1_v, wd2_v, ob0, ob1, semA, semB):
    c = lax.axis_index("c")
    s = lax.axis_index("s")
    pltpu.sync_copy(u3.at[c, s], uidx)
    pltpu.sync_copy(v3.at[c, s], vidx)
    pltpu.sync_copy(bd1, bd1_v)
    pltpu.sync_copy(wd2, wd2_v)
    bd1r = [bd1_v[pl.ds(r * L, L)] for r in range(D // L)]
    wd2r = [wd2_v[pl.ds(r * L, L)] for r in range(D // L)]

    def compute(j, ra, rb, ob):
        def edge(e, carry):
            acc = _zero16()
            for r in range(D // L):
                g = ra[e, pl.ds(r * L, L)] + rb[e, pl.ds(r * L, L)] + bd1r[r]
                acc = acc + jnp.maximum(g, 0.0) * wd2r[r]
            ob[e] = acc
            return carry
        lax.fori_loop(0, EB, edge, 0)
        pltpu.sync_copy(ob, out.at[c, s, j])

    pltpu.async_copy(zwu.at[uidx.at[0]], ra0, semA).wait()
    pltpu.async_copy(zwv.at[vidx.at[0]], rb0, semB).wait()

    def body(j, carry):
        b = 2 * j
        na = pltpu.async_copy(zwu.at[uidx.at[b + 1]], ra1, semA)
        nb = pltpu.async_copy(zwv.at[vidx.at[b + 1]], rb1, semB)
        compute(b, ra0, rb0, ob0)
        na.wait()
        nb.wait()
        ca = pltpu.async_copy(zwu.at[uidx.at[b + 2]], ra0, semA)
        cb = pltpu.async_copy(zwv.at[vidx.at[b + 2]], rb0, semB)
        compute(b + 1, ra1, rb1, ob1)
        ca.wait()
        cb.wait()
        return carry

    lax.fori_loop(0, (DEC_NB - 2) // 2, body, 0)
    na = pltpu.async_copy(zwu.at[uidx.at[DEC_NB - 1]], ra1, semA)
    nb = pltpu.async_copy(zwv.at[vidx.at[DEC_NB - 1]], rb1, semB)
    compute(DEC_NB - 2, ra0, rb0, ob0)
    na.wait()
    nb.wait()
    compute(DEC_NB - 1, ra1, rb1, ob1)


_BR = 512
_GRID = (N + _BR - 1) // _BR


def _mm0_body(x_ref, w_ref, o_ref):
    o_ref[...] = jnp.dot(x_ref[...], w_ref[...], preferred_element_type=_f32,
                         precision=lax.Precision.HIGHEST)


def _scale_body(cnt_ref, xw_ref, hs_ref, dis_ref):
    deg = cnt_ref[0] + cnt_ref[1] + 1.0
    dis = lax.rsqrt(deg)
    hs_ref[...] = xw_ref[...] * dis[:, None]
    dis_ref[...] = dis


def _mm2_body(acc_ref, hs_ref, dis_ref, b_ref, w_ref, out_ref, *, relu_in):
    dis = dis_ref[...]
    z = dis[:, None] * (acc_ref[0] + acc_ref[1] + hs_ref[...]) + b_ref[...][None, :]
    if relu_in:
        z = jnp.maximum(z, 0.0)
    h = jnp.dot(z, w_ref[...], preferred_element_type=_f32,
                precision=lax.Precision.HIGHEST)
    out_ref[...] = h * dis[:, None]


def _mm3_body(acc_ref, hs_ref, dis_ref, b_ref, w_ref, u_ref, v_ref):
    dis = dis_ref[...]
    z = dis[:, None] * (acc_ref[0] + acc_ref[1] + hs_ref[...]) + b_ref[...][None, :]
    w = w_ref[...]
    u_ref[...] = jnp.dot(z, w[:D], preferred_element_type=_f32,
                         precision=lax.Precision.HIGHEST)
    v_ref[...] = jnp.dot(z, w[D:], preferred_element_type=_f32,
                         precision=lax.Precision.HIGHEST)


def _tc_mm0(x, w):
    return pl.pallas_call(
        _mm0_body,
        grid=(_GRID,),
        in_specs=[
            pl.BlockSpec((_BR, D), lambda i: (i, 0)),
            pl.BlockSpec((D, D), lambda i: (0, 0)),
        ],
        out_specs=pl.BlockSpec((_BR, D), lambda i: (i, 0)),
        out_shape=jax.ShapeDtypeStruct((N, D), _f32),
    )(x, w)


def _tc_scale(cnt, xw):
    return pl.pallas_call(
        _scale_body,
        grid=(_GRID,),
        in_specs=[
            pl.BlockSpec((2, _BR), lambda i: (0, i)),
            pl.BlockSpec((_BR, D), lambda i: (i, 0)),
        ],
        out_specs=[
            pl.BlockSpec((_BR, D), lambda i: (i, 0)),
            pl.BlockSpec((_BR,), lambda i: (i,)),
        ],
        out_shape=[
            jax.ShapeDtypeStruct((N, D), _f32),
            jax.ShapeDtypeStruct((N,), _f32),
        ],
    )(cnt, xw)


def _tc_mm2(acc, hs, dis, b, w, relu_in):
    return pl.pallas_call(
        functools.partial(_mm2_body, relu_in=relu_in),
        grid=(_GRID,),
        in_specs=[
            pl.BlockSpec((2, _BR, D), lambda i: (0, i, 0)),
            pl.BlockSpec((_BR, D), lambda i: (i, 0)),
            pl.BlockSpec((_BR,), lambda i: (i,)),
            pl.BlockSpec((D,), lambda i: (0,)),
            pl.BlockSpec((D, D), lambda i: (0, 0)),
        ],
        out_specs=pl.BlockSpec((_BR, D), lambda i: (i, 0)),
        out_shape=jax.ShapeDtypeStruct((N, D), _f32),
    )(acc, hs, dis, b, w)


def _tc_mm3(acc, hs, dis, b, w):
    return pl.pallas_call(
        _mm3_body,
        grid=(_GRID,),
        in_specs=[
            pl.BlockSpec((2, _BR, D), lambda i: (0, i, 0)),
            pl.BlockSpec((_BR, D), lambda i: (i, 0)),
            pl.BlockSpec((_BR,), lambda i: (i,)),
            pl.BlockSpec((D,), lambda i: (0,)),
            pl.BlockSpec((2 * D, D), lambda i: (0, 0)),
        ],
        out_specs=[
            pl.BlockSpec((_BR, D), lambda i: (i, 0)),
            pl.BlockSpec((_BR, D), lambda i: (i, 0)),
        ],
        out_shape=[
            jax.ShapeDtypeStruct((N, D), _f32),
            jax.ShapeDtypeStruct((N, D), _f32),
        ],
    )(acc, hs, dis, b, w)


_RB = 8192
_RGRID = (EDEC + _RB - 1) // _RB


def _red_body(m_ref, b_ref, o_ref):
    o_ref[...] = jnp.sum(m_ref[...], axis=1) + b_ref[...][0]


def _tc_red(m, bd2):
    return pl.pallas_call(
        _red_body,
        grid=(_RGRID,),
        in_specs=[
            pl.BlockSpec((_RB, L), lambda i: (i, 0)),
            pl.BlockSpec((1,), lambda i: (0,)),
        ],
        out_specs=pl.BlockSpec((_RB,), lambda i: (i,)),
        out_shape=jax.ShapeDtypeStruct((EDEC,), _f32),
    )(m, bd2)


def kernel(x, edge_index, pos_edges, neg_edges, W1, b1, W2, b2, Wd1, bd1, Wd2, bd2):
    ei = edge_index.astype(jnp.int32)
    src3 = ei[0].reshape(NC, NS, S_PASS, S_CHUNK, SB)
    dst3 = ei[1].reshape(NC, NS, S_PASS, S_CHUNK, SB)
    dst3d = ei[1].reshape(NC, NS, E_NB, EB)
    u3 = jnp.concatenate([pos_edges[0], neg_edges[0]]).astype(jnp.int32)
    v3 = jnp.concatenate([pos_edges[1], neg_edges[1]]).astype(jnp.int32)
    u3 = u3.reshape(NC, NS, DEC_NB, EB)
    v3 = v3.reshape(NC, NS, DEC_NB, EB)

    cnt = _deg(dst3d)[:, 0, :]
    xw1 = _tc_mm0(x, W1)
    hs1, dis = _tc_scale(cnt, xw1)
    acc1 = _scatter(hs1, src3, dst3)
    hs2 = _tc_mm2(acc1, hs1, dis, b1, W2, relu_in=True)
    acc2 = _scatter(hs2, src3, dst3)
    zwu, zwv = _tc_mm3(acc2, hs2, dis, b2, Wd1)

    wd2 = Wd2.reshape(D)
    outg = _dec(zwu, zwv, u3, v3, bd1, wd2)
    return _tc_red(outg.reshape(EDEC, L), bd2)


# speedup vs baseline: 23.9974x; 1.3895x over previous
import functools

import jax
import jax.numpy as jnp
from jax import lax
from jax.experimental import pallas as pl
from jax.experimental.pallas import tpu as pltpu
from jax.experimental.pallas import tpu_sc as plsc

N = 10000
D = 128
E = 320000
EDEC = 200000

NC, NS, L = 2, 16, 16
NW = NC * NS

EB = 125
E_NB = E // (NW * EB)
DEC_NB = EDEC // (NW * EB)

SB = 80
S_NB = E // (NW * SB)
S_PASS = 5
S_CHUNK = S_NB // S_PASS

NPAD = 10240
STRIPE = NPAD // NS
DSTRIPE = NPAD // NS

_MESH = plsc.VectorSubcoreMesh(
    core_axis_name="c", subcore_axis_name="s", num_cores=NC, num_subcores=NS)

_f32 = jnp.float32


def _zero16():
    return jnp.zeros((L,), _f32)


@functools.partial(
    pl.kernel,
    out_type=jax.ShapeDtypeStruct((NC, 1, NPAD), _f32),
    mesh=_MESH,
    scratch_types=[
        pltpu.VMEM((E_NB, EB), jnp.int32),
        pltpu.VMEM((128,), _f32),
        pltpu.VMEM((DSTRIPE,), _f32),
        pltpu.VMEM_SHARED((NPAD,), _f32),
    ],
)
def _deg(dst3, out, idx_v, ones_v, zstripe, acc_sh):
    c = lax.axis_index("c")
    s = lax.axis_index("s")
    for k in range(8):
        ones_v[pl.ds(k * L, L)] = jnp.full((L,), 1.0, _f32)
    for k in range(DSTRIPE // L):
        zstripe[pl.ds(k * L, L)] = _zero16()
    pltpu.sync_copy(dst3.at[c, s], idx_v)
    pltpu.sync_copy(zstripe, acc_sh.at[pl.ds(s * DSTRIPE, DSTRIPE)])
    plsc.subcore_barrier()

    def body(j, carry):
        pltpu.sync_copy(ones_v.at[pl.ds(0, EB)], acc_sh.at[idx_v.at[j]], add=True)
        return carry

    lax.fori_loop(0, E_NB, body, 0)
    plsc.subcore_barrier()
    pltpu.sync_copy(acc_sh.at[pl.ds(s * DSTRIPE, DSTRIPE)],
                    out.at[c, 0, pl.ds(s * DSTRIPE, DSTRIPE)])


@functools.partial(
    pl.kernel,
    out_type=jax.ShapeDtypeStruct((NC, NPAD, D), _f32),
    mesh=_MESH,
    scratch_types=[
        pltpu.VMEM((S_CHUNK, SB), jnp.int32),
        pltpu.VMEM((S_CHUNK, SB), jnp.int32),
        pltpu.VMEM((SB, D), _f32),
        pltpu.VMEM((SB, D), _f32),
        pltpu.VMEM((SB, D), _f32),
        pltpu.VMEM_SHARED((NPAD, D), _f32),
        pltpu.SemaphoreType.DMA,
        pltpu.SemaphoreType.DMA,
        pltpu.SemaphoreType.DMA,
        pltpu.SemaphoreType.DMA,
        pltpu.SemaphoreType.DMA,
        pltpu.SemaphoreType.DMA,
    ],
)
def _scatter(hs, src3, dst3, out, sidx, didx, b0, b1, b2, acc_sh,
             gs0, gs1, gs2, ss0, ss1, ss2):
    c = lax.axis_index("c")
    s = lax.axis_index("s")
    bufs = (b0, b1, b2)
    gsems = (gs0, gs1, gs2)
    ssems = (ss0, ss1, ss2)
    nb = S_CHUNK

    def zrow(i, carry):
        for r in range(D // L):
            b0[i, pl.ds(r * L, L)] = _zero16()
        return carry

    lax.fori_loop(0, SB, zrow, 0)
    for t in range(STRIPE // SB):
        pltpu.sync_copy(b0, acc_sh.at[pl.ds(s * STRIPE + t * SB, SB)])
    plsc.subcore_barrier()

    def wait_gather(t, b):
        pltpu.make_async_copy(hs.at[sidx.at[t]], bufs[b], gsems[b]).wait()

    def wait_scatter(b):
        pltpu.make_async_copy(bufs[b], acc_sh.at[didx.at[0]], ssems[b]).wait()

    def issue_gather(t, b):
        pltpu.async_copy(hs.at[sidx.at[t]], bufs[b], gsems[b])

    def issue_scatter(t, b):
        pltpu.async_copy(bufs[b], acc_sh.at[didx.at[t]], ssems[b], add=True)

    def slot(t, b, swait):
        wait_gather(t, b)
        issue_scatter(t, b)
        if swait:
            wait_scatter((b + 2) % 3)
        issue_gather(t + 2, (b + 2) % 3)

    for p in range(S_PASS):
        pltpu.sync_copy(src3.at[c, s, p], sidx)
        pltpu.sync_copy(dst3.at[c, s, p], didx)
        issue_gather(0, 0)
        issue_gather(1, 1)
        slot(0, 0, False)

        def body(j, carry):
            t = 3 * j + 1
            slot(t, 1, True)
            slot(t + 1, 2, True)
            slot(t + 2, 0, True)
            return carry

        lax.fori_loop(0, (nb - 4) // 3, body, 0)
        t0 = nb - 3
        slot(t0, 1, True)
        wait_gather(t0 + 1, 2)
        issue_scatter(t0 + 1, 2)
        wait_gather(t0 + 2, 0)
        issue_scatter(t0 + 2, 0)
        wait_scatter(1)
        wait_scatter(2)
        wait_scatter(0)

    plsc.subcore_barrier()
    pltpu.sync_copy(acc_sh.at[pl.ds(s * STRIPE, STRIPE)],
                    out.at[c, pl.ds(s * STRIPE, STRIPE)])


@functools.partial(
    pl.kernel,
    out_type=jax.ShapeDtypeStruct((NC, NS, DEC_NB, 128), _f32),
    mesh=_MESH,
    scratch_types=[
        pltpu.VMEM((DEC_NB, EB), jnp.int32),
        pltpu.VMEM((DEC_NB, EB), jnp.int32),
        pltpu.VMEM((128, D), _f32),
        pltpu.VMEM((128, D), _f32),
        pltpu.VMEM((128, D), _f32),
        pltpu.VMEM((128, D), _f32),
        pltpu.VMEM((D,), _f32),
        pltpu.VMEM((D,), _f32),
        pltpu.VMEM((L,), _f32),
        pltpu.VMEM((DEC_NB, 128), _f32),
        pltpu.SemaphoreType.DMA,
        pltpu.SemaphoreType.DMA,
    ],
)
def _dec(zwu, zwv, u3, v3, bd1, wd2, bd2, out,
         uidx, vidx, ra0, rb0, ra1, rb1, bd1_v, wd2_v, bd2_v, ob,
         semA, semB):
    c = lax.axis_index("c")
    s = lax.axis_index("s")
    pltpu.sync_copy(u3.at[c, s], uidx)
    pltpu.sync_copy(v3.at[c, s], vidx)
    pltpu.sync_copy(bd1, bd1_v)
    pltpu.sync_copy(wd2, wd2_v)
    pltpu.sync_copy(bd2, bd2_v)
    bd1r = [bd1_v[pl.ds(r * L, L)] for r in range(D // L)]
    wd2r = [wd2_v[pl.ds(r * L, L)] for r in range(D // L)]
    bd2s = bd2_v[pl.ds(0, L)]
    lanes = lax.iota(jnp.int32, L)
    perms = [(lanes + sh) % L for sh in (8, 4, 2, 1)]

    def compute(j, ra, rb):
        for g in range(128 // L):
            def edge(i, ovec):
                e = g * L + i
                acc = _zero16()
                for r in range(D // L):
                    h = ra[e, pl.ds(r * L, L)] + rb[e, pl.ds(r * L, L)] + bd1r[r]
                    acc = acc + jnp.maximum(h, 0.0) * wd2r[r]
                for p in perms:
                    acc = acc + acc.at[p].get(mode="promise_in_bounds")
                return jnp.where(lanes == i, acc, ovec)
            ovec = lax.fori_loop(0, L, edge, _zero16())
            ob[j, pl.ds(g * L, L)] = ovec + bd2s

    pltpu.async_copy(zwu.at[uidx.at[0]], ra0.at[pl.ds(0, EB)], semA).wait()
    pltpu.async_copy(zwv.at[vidx.at[0]], rb0.at[pl.ds(0, EB)], semB).wait()

    def body(j, carry):
        b = 2 * j
        na = pltpu.async_copy(zwu.at[uidx.at[b + 1]], ra1.at[pl.ds(0, EB)], semA)
        nb = pltpu.async_copy(zwv.at[vidx.at[b + 1]], rb1.at[pl.ds(0, EB)], semB)
        compute(b, ra0, rb0)
        na.wait()
        nb.wait()
        ca = pltpu.async_copy(zwu.at[uidx.at[b + 2]], ra0.at[pl.ds(0, EB)], semA)
        cb = pltpu.async_copy(zwv.at[vidx.at[b + 2]], rb0.at[pl.ds(0, EB)], semB)
        compute(b + 1, ra1, rb1)
        ca.wait()
        cb.wait()
        return carry

    lax.fori_loop(0, (DEC_NB - 2) // 2, body, 0)
    na = pltpu.async_copy(zwu.at[uidx.at[DEC_NB - 1]], ra1.at[pl.ds(0, EB)], semA)
    nb = pltpu.async_copy(zwv.at[vidx.at[DEC_NB - 1]], rb1.at[pl.ds(0, EB)], semB)
    compute(DEC_NB - 2, ra0, rb0)
    na.wait()
    nb.wait()
    compute(DEC_NB - 1, ra1, rb1)
    pltpu.sync_copy(ob, out.at[c, s])


_BR = 512
_GRID = (N + _BR - 1) // _BR


def _mm0_body(x_ref, w_ref, o_ref):
    o_ref[...] = jnp.dot(x_ref[...], w_ref[...], preferred_element_type=_f32,
                         precision=lax.Precision.HIGHEST)


def _scale_body(cnt_ref, xw_ref, hs_ref, dis_ref):
    deg = cnt_ref[0] + cnt_ref[1] + 1.0
    dis = lax.rsqrt(deg)
    hs_ref[...] = xw_ref[...] * dis[:, None]
    dis_ref[...] = dis


def _mm2_body(acc_ref, hs_ref, dis_ref, b_ref, w_ref, out_ref, *, relu_in):
    dis = dis_ref[...]
    z = dis[:, None] * (acc_ref[0] + acc_ref[1] + hs_ref[...]) + b_ref[...][None, :]
    if relu_in:
        z = jnp.maximum(z, 0.0)
    h = jnp.dot(z, w_ref[...], preferred_element_type=_f32,
                precision=lax.Precision.HIGHEST)
    out_ref[...] = h * dis[:, None]


def _mm3_body(acc_ref, hs_ref, dis_ref, b_ref, w_ref, u_ref, v_ref):
    dis = dis_ref[...]
    z = dis[:, None] * (acc_ref[0] + acc_ref[1] + hs_ref[...]) + b_ref[...][None, :]
    w = w_ref[...]
    u_ref[...] = jnp.dot(z, w[:D], preferred_element_type=_f32,
                         precision=lax.Precision.HIGHEST)
    v_ref[...] = jnp.dot(z, w[D:], preferred_element_type=_f32,
                         precision=lax.Precision.HIGHEST)


def _tc_mm0(x, w):
    return pl.pallas_call(
        _mm0_body,
        grid=(_GRID,),
        in_specs=[
            pl.BlockSpec((_BR, D), lambda i: (i, 0)),
            pl.BlockSpec((D, D), lambda i: (0, 0)),
        ],
        out_specs=pl.BlockSpec((_BR, D), lambda i: (i, 0)),
        out_shape=jax.ShapeDtypeStruct((N, D), _f32),
    )(x, w)


def _tc_scale(cnt, xw):
    return pl.pallas_call(
        _scale_body,
        grid=(_GRID,),
        in_specs=[
            pl.BlockSpec((2, _BR), lambda i: (0, i)),
            pl.BlockSpec((_BR, D), lambda i: (i, 0)),
        ],
        out_specs=[
            pl.BlockSpec((_BR, D), lambda i: (i, 0)),
            pl.BlockSpec((_BR,), lambda i: (i,)),
        ],
        out_shape=[
            jax.ShapeDtypeStruct((N, D), _f32),
            jax.ShapeDtypeStruct((N,), _f32),
        ],
    )(cnt, xw)


def _tc_mm2(acc, hs, dis, b, w, relu_in):
    return pl.pallas_call(
        functools.partial(_mm2_body, relu_in=relu_in),
        grid=(_GRID,),
        in_specs=[
            pl.BlockSpec((2, _BR, D), lambda i: (0, i, 0)),
            pl.BlockSpec((_BR, D), lambda i: (i, 0)),
            pl.BlockSpec((_BR,), lambda i: (i,)),
            pl.BlockSpec((D,), lambda i: (0,)),
            pl.BlockSpec((D, D), lambda i: (0, 0)),
        ],
        out_specs=pl.BlockSpec((_BR, D), lambda i: (i, 0)),
        out_shape=jax.ShapeDtypeStruct((N, D), _f32),
    )(acc, hs, dis, b, w)


def _tc_mm3(acc, hs, dis, b, w):
    return pl.pallas_call(
        _mm3_body,
        grid=(_GRID,),
        in_specs=[
            pl.BlockSpec((2, _BR, D), lambda i: (0, i, 0)),
            pl.BlockSpec((_BR, D), lambda i: (i, 0)),
            pl.BlockSpec((_BR,), lambda i: (i,)),
            pl.BlockSpec((D,), lambda i: (0,)),
            pl.BlockSpec((2 * D, D), lambda i: (0, 0)),
        ],
        out_specs=[
            pl.BlockSpec((_BR, D), lambda i: (i, 0)),
            pl.BlockSpec((_BR, D), lambda i: (i, 0)),
        ],
        out_shape=[
            jax.ShapeDtypeStruct((N, D), _f32),
            jax.ShapeDtypeStruct((N, D), _f32),
        ],
    )(acc, hs, dis, b, w)


def kernel(x, edge_index, pos_edges, neg_edges, W1, b1, W2, b2, Wd1, bd1, Wd2, bd2):
    ei = edge_index.astype(jnp.int32)
    src3 = ei[0].reshape(NC, NS, S_PASS, S_CHUNK, SB)
    dst3 = ei[1].reshape(NC, NS, S_PASS, S_CHUNK, SB)
    dst3d = ei[1].reshape(NC, NS, E_NB, EB)
    u3 = jnp.concatenate([pos_edges[0], neg_edges[0]]).astype(jnp.int32)
    v3 = jnp.concatenate([pos_edges[1], neg_edges[1]]).astype(jnp.int32)
    u3 = u3.reshape(NC, NS, DEC_NB, EB)
    v3 = v3.reshape(NC, NS, DEC_NB, EB)

    cnt = _deg(dst3d)[:, 0, :]
    xw1 = _tc_mm0(x, W1)
    hs1, dis = _tc_scale(cnt, xw1)
    acc1 = _scatter(hs1, src3, dst3)
    hs2 = _tc_mm2(acc1, hs1, dis, b1, W2, relu_in=True)
    acc2 = _scatter(hs2, src3, dst3)
    zwu, zwv = _tc_mm3(acc2, hs2, dis, b2, Wd1)

    wd2 = Wd2.reshape(D)
    bd2v = jnp.broadcast_to(bd2, (L,))
    outg = _dec(zwu, zwv, u3, v3, bd1, wd2, bd2v)
    return outg[..., :EB].reshape(EDEC)
